# baseline (speedup 1.0000x reference)
import jax
import jax.numpy as jnp
import numpy as np
from jax import lax
from jax.experimental import pallas as pl
from jax.experimental.pallas import tpu as pltpu
from jax.experimental.pallas import tpu_sc as plsc

_NV = 128
_NE = 8
_NS = 7
_EXP = 1 + _NS + _NE
_B = 4
_T = 8192
_TOTAL = _NV * _EXP
_C = 8
_NW = 32
_TPW = _T // _NW
_NCH = _TPW // _C
_STW = _NV * _NS
_ENW = _NV * _NE
_ST_OFF = _B * _NV
_EN_OFF = _ST_OFF + _STW
_SRCW = _EN_OFF + _ENW
_HALF = _TOTAL // 2


def _perm_table() -> np.ndarray:
    p = np.zeros(_TOTAL, np.int32)
    for v in range(_NV):
        p[v * _EXP] = v
        for r in range(1, 1 + _NS):
            p[v * _EXP + r] = _ST_OFF + v * _NS + (r - 1)
        for r in range(1 + _NS, _EXP):
            p[v * _EXP + r] = _EN_OFF + v * _NE + (r - 1 - _NS)
    return p


def _sc_body(inp_ref, enc_ref, st_ref, perm_ref, out_ref,
             perm_v, src0, src1, half0, half1,
             sem_i0, sem_i1, sem_o0, sem_o1):
    c = lax.axis_index("c")
    s = lax.axis_index("s")
    wid = s * 2 + c
    t0 = wid * _TPW
    srcs = (src0, src1)
    halves = (half0, half1)
    sems_i = (sem_i0, sem_i1)
    sems_o = (sem_o0, sem_o1)
    pltpu.sync_copy(perm_ref, perm_v)
    iota = lax.iota(jnp.int32, 16)
    fix_cols = iota * jnp.int32(16)

    def issue_in(k, p):
        tc = t0 + k * _C
        for b in range(_B):
            pltpu.async_copy(inp_ref.at[b, pl.ds(tc, _C), :],
                             srcs[p].at[:, pl.ds(b * _NV, _NV)],
                             sems_i[p])
        pltpu.async_copy(st_ref.at[pl.ds(tc, _C), :],
                         srcs[p].at[:, pl.ds(_ST_OFF, _STW)],
                         sems_i[p])
        pltpu.async_copy(enc_ref.at[pl.ds(tc, _C), :],
                         srcs[p].at[:, pl.ds(_EN_OFF, _ENW)],
                         sems_i[p])

    def drain_in(p):
        for b in range(_B):
            pltpu.make_async_copy(inp_ref.at[0, pl.ds(0, _C), :],
                                  srcs[p].at[:, pl.ds(b * _NV, _NV)],
                                  sems_i[p]).wait()
        pltpu.make_async_copy(st_ref.at[pl.ds(0, _C), :],
                              srcs[p].at[:, pl.ds(_ST_OFF, _STW)],
                              sems_i[p]).wait()
        pltpu.make_async_copy(enc_ref.at[pl.ds(0, _C), :],
                              srcs[p].at[:, pl.ds(_EN_OFF, _ENW)],
                              sems_i[p]).wait()

    def issue_out(k, h):
        tc = t0 + k * _C
        for b in range(_B):
            pltpu.async_copy(halves[h].at[pl.ds(b * _C, _C), :],
                             out_ref.at[b, pl.ds(tc, _C),
                                        pl.ds(h * _HALF, _HALF)],
                             sems_o[h])

    def drain_out(h):
        for b in range(_B):
            pltpu.make_async_copy(halves[h].at[pl.ds(b * _C, _C), :],
                                  out_ref.at[0, pl.ds(0, _C), pl.ds(0, _HALF)],
                                  sems_o[h]).wait()

    def assemble(p, h):
        @plsc.parallel_loop(0, _NV // 2, unroll=8)
        def _grp(g):
            cols = perm_v[pl.ds((h * (_NV // 2) + g) * 16, 16)]
            for j in range(_C):
                rows = jnp.full((16,), j, jnp.int32)
                vals = plsc.load_gather(srcs[p], [rows, cols])
                for b in range(_B):
                    halves[h][b * _C + j, pl.ds(g * 16, 16)] = vals

        for j in range(_C):
            for b in range(1, _B):
                for gi in range(4):
                    v0 = h * (_NV // 2) + gi * 16
                    vals = srcs[p][j, pl.ds(b * _NV + v0, 16)]
                    plsc.store_scatter(
                        halves[h],
                        [jnp.full((16,), b * _C + j, jnp.int32),
                         fix_cols + jnp.int32(gi * 16 * _EXP)],
                        vals)

    issue_in(0, 0)

    @pl.loop(0, _NCH, step=2)
    def _chunk(k0):
        for p in range(2):
            k = k0 + p

            @pl.when(k + 1 < _NCH)
            def _():
                issue_in(k + 1, 1 - p)

            drain_in(p)
            for h in range(2):
                @pl.when(k >= 1)
                def _():
                    drain_out(h)

                assemble(p, h)
                issue_out(k, h)

    drain_out(0)
    drain_out(1)


def kernel(inp, variable_encodings, static_channels):
    perm = jnp.asarray(_perm_table())
    mesh = plsc.VectorSubcoreMesh(core_axis_name="c", subcore_axis_name="s",
                                  num_cores=2, num_subcores=16)
    f = pl.kernel(
        _sc_body,
        out_type=jax.ShapeDtypeStruct((_B, _T, _TOTAL), jnp.float32),
        mesh=mesh,
        compiler_params=pltpu.CompilerParams(needs_layout_passes=False,
                                             use_tc_tiling_on_sc=True),
        scratch_types=[
            pltpu.VMEM((_TOTAL,), jnp.int32),
            pltpu.VMEM((_C, _SRCW), jnp.float32),
            pltpu.VMEM((_C, _SRCW), jnp.float32),
            pltpu.VMEM((_B * _C, _HALF), jnp.float32),
            pltpu.VMEM((_B * _C, _HALF), jnp.float32),
            pltpu.SemaphoreType.DMA,
            pltpu.SemaphoreType.DMA,
            pltpu.SemaphoreType.DMA,
            pltpu.SemaphoreType.DMA,
        ],
    )
    return f(inp, variable_encodings, static_channels, perm)


# speedup vs baseline: 1.0213x; 1.0213x over previous
import jax
import jax.numpy as jnp
import numpy as np
from jax import lax
from jax.experimental import pallas as pl
from jax.experimental.pallas import tpu as pltpu
from jax.experimental.pallas import tpu_sc as plsc

_NV = 128
_NE = 8
_NS = 7
_EXP = 1 + _NS + _NE
_B = 4
_T = 8192
_TOTAL = _NV * _EXP
_C = 8
_NW = 32
_TPW = _T // _NW
_NCH = _TPW // _C
_STW = _NV * _NS
_ENW = _NV * _NE
_ST_OFF = _B * _NV
_EN_OFF = _ST_OFF + _STW
_SRCW = _EN_OFF + _ENW
_HALF = _TOTAL // 2


def _perm_table() -> np.ndarray:
    p = np.zeros(_TOTAL, np.int32)
    for v in range(_NV):
        p[v * _EXP] = v
        for r in range(1, 1 + _NS):
            p[v * _EXP + r] = _ST_OFF + v * _NS + (r - 1)
        for r in range(1 + _NS, _EXP):
            p[v * _EXP + r] = _EN_OFF + v * _NE + (r - 1 - _NS)
    return p


def _sc_body(inp_ref, enc_ref, st_ref, perm_ref, out_ref,
             perm_v, src0, src1, half0, half1,
             sem_i0, sem_i1, sem_o0, sem_o1):
    c = lax.axis_index("c")
    s = lax.axis_index("s")
    wid = s * 2 + c
    t0 = wid * _TPW
    srcs = (src0, src1)
    halves = (half0, half1)
    sems_i = (sem_i0, sem_i1)
    sems_o = (sem_o0, sem_o1)
    pltpu.sync_copy(perm_ref, perm_v)
    iota = lax.iota(jnp.int32, 16)
    fix_cols = iota * jnp.int32(16)

    def issue_in(k, p):
        tc = t0 + k * _C
        for b in range(_B):
            pltpu.async_copy(inp_ref.at[b, pl.ds(tc, _C), :],
                             srcs[p].at[:, pl.ds(b * _NV, _NV)],
                             sems_i[p])
        pltpu.async_copy(st_ref.at[pl.ds(tc, _C), :],
                         srcs[p].at[:, pl.ds(_ST_OFF, _STW)],
                         sems_i[p])
        pltpu.async_copy(enc_ref.at[pl.ds(tc, _C), :],
                         srcs[p].at[:, pl.ds(_EN_OFF, _ENW)],
                         sems_i[p])

    def drain_in(p):
        for b in range(_B):
            pltpu.make_async_copy(inp_ref.at[0, pl.ds(0, _C), :],
                                  srcs[p].at[:, pl.ds(b * _NV, _NV)],
                                  sems_i[p]).wait()
        pltpu.make_async_copy(st_ref.at[pl.ds(0, _C), :],
                              srcs[p].at[:, pl.ds(_ST_OFF, _STW)],
                              sems_i[p]).wait()
        pltpu.make_async_copy(enc_ref.at[pl.ds(0, _C), :],
                              srcs[p].at[:, pl.ds(_EN_OFF, _ENW)],
                              sems_i[p]).wait()

    def issue_out(k, h):
        tc = t0 + k * _C
        for b in range(_B):
            pltpu.async_copy(halves[h].at[pl.ds(b * _C, _C), :],
                             out_ref.at[b, pl.ds(tc, _C),
                                        pl.ds(h * _HALF, _HALF)],
                             sems_o[h])

    def drain_out(h):
        for b in range(_B):
            pltpu.make_async_copy(halves[h].at[pl.ds(b * _C, _C), :],
                                  out_ref.at[0, pl.ds(0, _C), pl.ds(0, _HALF)],
                                  sems_o[h]).wait()

    def assemble(p, h):
        @plsc.parallel_loop(0, _NV // 2, unroll=4)
        def _grp(g):
            cols = perm_v[pl.ds((h * (_NV // 2) + g) * 16, 16)]
            for j in range(_C):
                rows = jnp.full((16,), j, jnp.int32)
                vals = plsc.load_gather(srcs[p], [rows, cols])
                for b in range(_B):
                    halves[h][b * _C + j, pl.ds(g * 16, 16)] = vals

        for j in range(_C):
            for b in range(1, _B):
                for gi in range(4):
                    v0 = h * (_NV // 2) + gi * 16
                    vals = srcs[p][j, pl.ds(b * _NV + v0, 16)]
                    plsc.store_scatter(
                        halves[h],
                        [jnp.full((16,), b * _C + j, jnp.int32),
                         fix_cols + jnp.int32(gi * 16 * _EXP)],
                        vals)

    issue_in(0, 0)

    @pl.loop(0, _NCH, step=2)
    def _chunk(k0):
        for p in range(2):
            k = k0 + p

            @pl.when(k + 1 < _NCH)
            def _():
                issue_in(k + 1, 1 - p)

            drain_in(p)
            for h in range(2):
                @pl.when(k >= 1)
                def _():
                    drain_out(h)

                assemble(p, h)
                issue_out(k, h)

    drain_out(0)
    drain_out(1)


def kernel(inp, variable_encodings, static_channels):
    perm = jnp.asarray(_perm_table())
    mesh = plsc.VectorSubcoreMesh(core_axis_name="c", subcore_axis_name="s",
                                  num_cores=2, num_subcores=16)
    f = pl.kernel(
        _sc_body,
        out_type=jax.ShapeDtypeStruct((_B, _T, _TOTAL), jnp.float32),
        mesh=mesh,
        compiler_params=pltpu.CompilerParams(needs_layout_passes=False,
                                             use_tc_tiling_on_sc=True),
        scratch_types=[
            pltpu.VMEM((_TOTAL,), jnp.int32),
            pltpu.VMEM((_C, _SRCW), jnp.float32),
            pltpu.VMEM((_C, _SRCW), jnp.float32),
            pltpu.VMEM((_B * _C, _HALF), jnp.float32),
            pltpu.VMEM((_B * _C, _HALF), jnp.float32),
            pltpu.SemaphoreType.DMA,
            pltpu.SemaphoreType.DMA,
            pltpu.SemaphoreType.DMA,
            pltpu.SemaphoreType.DMA,
        ],
    )
    return f(inp, variable_encodings, static_channels, perm)
